# baseline (speedup 1.0000x reference)
import functools

import jax
import jax.numpy as jnp
from jax import lax
from jax.experimental import pallas as pl
from jax.experimental.pallas import tpu as pltpu
from jax.experimental.pallas import tpu_sc as plsc

DIM = 128
GAMMA = 1.0
L = 16


def _sc_info():
    try:
        info = plsc.get_sparse_core_info()
        return info.num_cores, info.num_subcores
    except Exception:
        return 2, 16


@functools.lru_cache(maxsize=None)
def _build_sc(R):
    NC, NS = _sc_info()
    NW = NC * NS
    assert R % NW == 0
    n_per_w = R // NW
    CH = min(128, n_per_w)
    n_chunks = n_per_w // CH
    NBUF = min(2, n_chunks)
    NJ = DIM // L
    n_groups = CH // L

    mesh = plsc.VectorSubcoreMesh(core_axis_name="c", subcore_axis_name="s")

    row_buf = pltpu.VMEM((CH, DIM), jnp.float32)

    @functools.partial(
        pl.kernel,
        out_type=jax.ShapeDtypeStruct((NW, L), jnp.float32),
        mesh=mesh,
        compiler_params=pltpu.CompilerParams(needs_layout_passes=False),
        scratch_types=[
            pltpu.VMEM((n_per_w,), jnp.int32),
            pltpu.VMEM((n_per_w,), jnp.int32),
            pltpu.VMEM((n_per_w,), jnp.float32),
            pltpu.VMEM((8,), jnp.int32),
            pltpu.VMEM((8, DIM), jnp.float32),
            [row_buf] * NBUF,
            [row_buf] * NBUF,
            pltpu.VMEM((L, L), jnp.float32),
            [pltpu.SemaphoreType.DMA] * NBUF,
            [pltpu.SemaphoreType.DMA] * NBUF,
            pltpu.SemaphoreType.DMA,
            pltpu.SemaphoreType.DMA,
            pltpu.SemaphoreType.DMA,
        ],
    )
    def sc_kernel(table_hbm, pconf_hbm, aidx_hbm, pos_hbm, neg_hbm, out_hbm,
                  posidx_v, negidx_v, pconf_v, aidx_v, a_v,
                  posbufs, negbufs, comb_v, psems, nsems, sem_i, sem_j, sem_s):
        wid = lax.axis_index("s") * NC + lax.axis_index("c")
        base = wid * n_per_w

        ci = pltpu.async_copy(pos_hbm.at[pl.ds(base, n_per_w)], posidx_v, sem_i)
        cj = pltpu.async_copy(neg_hbm.at[pl.ds(base, n_per_w)], negidx_v, sem_j)
        ck = pltpu.async_copy(pconf_hbm.at[pl.ds(base, n_per_w)], pconf_v, sem_s)
        pltpu.sync_copy(aidx_hbm, aidx_v)
        ca = pltpu.async_copy(table_hbm.at[aidx_v], a_v, sem_s)

        def start_pos(c):
            b = c % NBUF
            return pltpu.async_copy(
                table_hbm.at[posidx_v.at[pl.ds(c * CH, CH)]], posbufs[b],
                psems[b])

        def start_neg(c):
            b = c % NBUF
            return pltpu.async_copy(
                table_hbm.at[negidx_v.at[pl.ds(c * CH, CH)]], negbufs[b],
                nsems[b])

        ci.wait()
        pend_p = {c: start_pos(c) for c in range(NBUF)}
        cj.wait()
        pend_n = {c: start_neg(c) for c in range(NBUF)}
        pending = {c: (pend_p[c], pend_n[c]) for c in range(NBUF)}
        ck.wait()
        ca.wait()
        a_sl = [a_v[0, pl.ds(L * j, L)] for j in range(NJ)]

        lane = jnp.arange(L, dtype=jnp.int32)
        loss_vec = jnp.zeros((L,), jnp.float32)
        for c in range(n_chunks):
            cp, cn = pending.pop(c)
            cp.wait()
            cn.wait()
            posrows_v = posbufs[c % NBUF]
            negrows_v = negbufs[c % NBUF]

            def group_body(g, lvec, posrows_v=posrows_v, negrows_v=negrows_v,
                           c=c):
                base_r = g * L
                for r in range(L):
                    rr = base_r + r
                    pa = jnp.abs(a_sl[0] - posrows_v[rr, pl.ds(0, L)])
                    pb = jnp.abs(a_sl[1] - posrows_v[rr, pl.ds(L, L)])
                    na = jnp.abs(a_sl[0] - negrows_v[rr, pl.ds(0, L)])
                    nb = jnp.abs(a_sl[1] - negrows_v[rr, pl.ds(L, L)])
                    for j in range(2, NJ, 2):
                        pa = pa + jnp.abs(
                            a_sl[j] - posrows_v[rr, pl.ds(L * j, L)])
                        pb = pb + jnp.abs(
                            a_sl[j + 1] - posrows_v[rr, pl.ds(L * (j + 1), L)])
                        na = na + jnp.abs(
                            a_sl[j] - negrows_v[rr, pl.ds(L * j, L)])
                        nb = nb + jnp.abs(
                            a_sl[j + 1] - negrows_v[rr, pl.ds(L * (j + 1), L)])
                    pcs = plsc.load_gather(
                        pconf_v, [jnp.full((L,), c * CH + rr, jnp.int32)])
                    comb_v[r] = pcs * (pa + pb) - (na + nb)
                colsum = plsc.load_gather(
                    comb_v, [lane, jnp.zeros((L,), jnp.int32)])
                for j in range(1, L):
                    colsum = colsum + plsc.load_gather(
                        comb_v, [lane, jnp.full((L,), j, jnp.int32)])
                return lvec + jnp.maximum(GAMMA + colsum, jnp.float32(0.0))

            loss_vec = lax.fori_loop(0, n_groups, group_body, loss_vec)
            if c + NBUF < n_chunks:
                pending[c + NBUF] = (start_pos(c + NBUF), start_neg(c + NBUF))

        comb_v[0] = loss_vec
        pltpu.sync_copy(comb_v.at[0], out_hbm.at[wid])

    return sc_kernel


def kernel(rel_emb, pconfi, rel_a, rel_pos, rel_neg):
    R = rel_pos.shape[0]
    sc = _build_sc(R)
    aidx = jnp.full((8,), rel_a, jnp.int32)
    partials = sc(rel_emb, pconfi, aidx,
                  rel_pos.astype(jnp.int32), rel_neg.astype(jnp.int32))
    return jnp.sum(partials)


# speedup vs baseline: 1.2149x; 1.2149x over previous
import functools

import jax
import jax.numpy as jnp
from jax import lax
from jax.experimental import pallas as pl
from jax.experimental.pallas import tpu as pltpu
from jax.experimental.pallas import tpu_sc as plsc

DIM = 128
GAMMA = 1.0
L = 16


def _sc_info():
    try:
        info = plsc.get_sparse_core_info()
        return info.num_cores, info.num_subcores
    except Exception:
        return 2, 16


@functools.lru_cache(maxsize=None)
def _build_sc(R):
    NC, NS = _sc_info()
    NW = NC * NS
    assert R % NW == 0
    n_per_w = R // NW
    CH = min(128, n_per_w)
    n_chunks = n_per_w // CH
    NBUF = min(2, n_chunks)
    NJ = DIM // L
    n_groups = CH // L

    mesh = plsc.VectorSubcoreMesh(core_axis_name="c", subcore_axis_name="s")

    row_buf = pltpu.VMEM((CH, DIM), jnp.float32)

    @functools.partial(
        pl.kernel,
        out_type=jax.ShapeDtypeStruct((NW, L), jnp.float32),
        mesh=mesh,
        compiler_params=pltpu.CompilerParams(needs_layout_passes=False),
        scratch_types=[
            pltpu.VMEM((n_per_w,), jnp.int32),
            pltpu.VMEM((n_per_w,), jnp.int32),
            pltpu.VMEM((n_per_w,), jnp.float32),
            pltpu.VMEM((8,), jnp.int32),
            pltpu.VMEM((8, DIM), jnp.float32),
            [row_buf] * NBUF,
            [row_buf] * NBUF,
            pltpu.VMEM((L,), jnp.float32),
            [pltpu.SemaphoreType.DMA] * NBUF,
            [pltpu.SemaphoreType.DMA] * NBUF,
            pltpu.SemaphoreType.DMA,
            pltpu.SemaphoreType.DMA,
            pltpu.SemaphoreType.DMA,
        ],
    )
    def sc_kernel(table_hbm, pconf_hbm, aidx_hbm, pos_hbm, neg_hbm, out_hbm,
                  posidx_v, negidx_v, pconf_v, aidx_v, a_v,
                  posbufs, negbufs, outv, psems, nsems, sem_i, sem_j, sem_s):
        wid = lax.axis_index("s") * NC + lax.axis_index("c")
        base = wid * n_per_w

        ci = pltpu.async_copy(pos_hbm.at[pl.ds(base, n_per_w)], posidx_v, sem_i)
        cj = pltpu.async_copy(neg_hbm.at[pl.ds(base, n_per_w)], negidx_v, sem_j)
        ck = pltpu.async_copy(pconf_hbm.at[pl.ds(base, n_per_w)], pconf_v, sem_s)
        pltpu.sync_copy(aidx_hbm, aidx_v)
        ca = pltpu.async_copy(table_hbm.at[aidx_v], a_v, sem_s)

        def start_pos(c):
            b = c % NBUF
            return pltpu.async_copy(
                table_hbm.at[posidx_v.at[pl.ds(c * CH, CH)]], posbufs[b],
                psems[b])

        def start_neg(c):
            b = c % NBUF
            return pltpu.async_copy(
                table_hbm.at[negidx_v.at[pl.ds(c * CH, CH)]], negbufs[b],
                nsems[b])

        ci.wait()
        cp0 = start_pos(0)
        cj.wait()
        cn0 = start_neg(0)
        pending = {0: (cp0, cn0)}
        ck.wait()
        ca.wait()
        a_sl = [a_v[0, pl.ds(L * j, L)] for j in range(NJ)]

        total = jnp.float32(0.0)
        for c in range(n_chunks):
            cp, cn = pending.pop(c)
            cp.wait()
            cn.wait()
            if c + 1 < n_chunks:
                pending[c + 1] = (start_pos(c + 1), start_neg(c + 1))
            posrows_v = posbufs[c % NBUF]
            negrows_v = negbufs[c % NBUF]

            def group_body(g, tot, posrows_v=posrows_v, negrows_v=negrows_v,
                           c=c):
                base_r = g * L
                pcg = pconf_v[pl.ds(c * CH + base_r, L)]
                for r in range(L):
                    rr = base_r + r
                    pacc = jnp.abs(a_sl[0] - posrows_v[rr, pl.ds(0, L)])
                    nacc = jnp.abs(a_sl[0] - negrows_v[rr, pl.ds(0, L)])
                    for j in range(1, NJ):
                        pacc = pacc + jnp.abs(
                            a_sl[j] - posrows_v[rr, pl.ds(L * j, L)])
                        nacc = nacc + jnp.abs(
                            a_sl[j] - negrows_v[rr, pl.ds(L * j, L)])
                    sp = jnp.sum(pacc)
                    sn = jnp.sum(nacc)
                    tot = tot + jnp.maximum(
                        GAMMA + pcg[r] * sp - sn, jnp.float32(0.0))
                return tot

            total = lax.fori_loop(0, n_groups, group_body, total)

        outv[...] = jnp.where(lax.iota(jnp.int32, L) == 0, total,
                              jnp.float32(0.0))
        pltpu.sync_copy(outv, out_hbm.at[wid])

    return sc_kernel


def kernel(rel_emb, pconfi, rel_a, rel_pos, rel_neg):
    R = rel_pos.shape[0]
    sc = _build_sc(R)
    aidx = jnp.full((8,), rel_a, jnp.int32)
    partials = sc(rel_emb, pconfi, aidx,
                  rel_pos.astype(jnp.int32), rel_neg.astype(jnp.int32))
    return jnp.sum(partials)


# speedup vs baseline: 1.2219x; 1.0058x over previous
import functools

import jax
import jax.numpy as jnp
from jax import lax
from jax.experimental import pallas as pl
from jax.experimental.pallas import tpu as pltpu
from jax.experimental.pallas import tpu_sc as plsc

DIM = 128
GAMMA = 1.0
L = 16


def _sc_info():
    try:
        info = plsc.get_sparse_core_info()
        return info.num_cores, info.num_subcores
    except Exception:
        return 2, 16


@functools.lru_cache(maxsize=None)
def _build_sc(R):
    NC, NS = _sc_info()
    NW = NC * NS
    assert R % NW == 0
    n_per_w = R // NW
    CH = min(128, n_per_w)
    n_chunks = n_per_w // CH
    NBUF = min(2, n_chunks)
    NJ = DIM // L
    n_groups = CH // L

    mesh = plsc.VectorSubcoreMesh(core_axis_name="c", subcore_axis_name="s")

    row_buf = pltpu.VMEM((CH, DIM), jnp.float32)

    @functools.partial(
        pl.kernel,
        out_type=jax.ShapeDtypeStruct((NW, L), jnp.float32),
        mesh=mesh,
        compiler_params=pltpu.CompilerParams(needs_layout_passes=False),
        scratch_types=[
            pltpu.VMEM((n_per_w,), jnp.int32),
            pltpu.VMEM((n_per_w,), jnp.int32),
            pltpu.VMEM((n_per_w,), jnp.float32),
            pltpu.VMEM((8,), jnp.int32),
            pltpu.VMEM((8, DIM), jnp.float32),
            [row_buf] * NBUF,
            [row_buf] * NBUF,
            pltpu.VMEM((L,), jnp.float32),
            [pltpu.SemaphoreType.DMA] * NBUF,
            [pltpu.SemaphoreType.DMA] * NBUF,
            pltpu.SemaphoreType.DMA,
            pltpu.SemaphoreType.DMA,
            pltpu.SemaphoreType.DMA,
        ],
    )
    def sc_kernel(table_hbm, pconf_hbm, aidx_hbm, pos_hbm, neg_hbm, out_hbm,
                  posidx_v, negidx_v, pconf_v, aidx_v, a_v,
                  posbufs, negbufs, outv, psems, nsems, sem_i, sem_j, sem_s):
        wid = lax.axis_index("s") * NC + lax.axis_index("c")
        base = wid * n_per_w

        ci = pltpu.async_copy(pos_hbm.at[pl.ds(base, n_per_w)], posidx_v, sem_i)
        cj = pltpu.async_copy(neg_hbm.at[pl.ds(base, n_per_w)], negidx_v, sem_j)
        ck = pltpu.async_copy(pconf_hbm.at[pl.ds(base, n_per_w)], pconf_v, sem_s)
        pltpu.sync_copy(aidx_hbm, aidx_v)
        ca = pltpu.async_copy(table_hbm.at[aidx_v], a_v, sem_s)

        def start_pos(c):
            b = c % NBUF
            return pltpu.async_copy(
                table_hbm.at[posidx_v.at[pl.ds(c * CH, CH)]], posbufs[b],
                psems[b])

        def start_neg(c):
            b = c % NBUF
            return pltpu.async_copy(
                table_hbm.at[negidx_v.at[pl.ds(c * CH, CH)]], negbufs[b],
                nsems[b])

        ci.wait()
        cp0 = start_pos(0)
        cj.wait()
        cn0 = start_neg(0)
        pending = {0: (cp0, cn0)}
        ck.wait()
        ca.wait()
        a_sl = [a_v[0, pl.ds(L * j, L)] for j in range(NJ)]

        total = jnp.float32(0.0)
        for c in range(n_chunks):
            cp, cn = pending.pop(c)
            cp.wait()
            cn.wait()
            if c + 1 < n_chunks:
                pending[c + 1] = (start_pos(c + 1), start_neg(c + 1))
            posrows_v = posbufs[c % NBUF]
            negrows_v = negbufs[c % NBUF]

            def group_body(g, tot, posrows_v=posrows_v, negrows_v=negrows_v,
                           c=c):
                base_r = g * L
                pcg = pconf_v[pl.ds(c * CH + base_r, L)]
                for r in range(L):
                    rr = base_r + r
                    pacc = jnp.abs(a_sl[0] - posrows_v[rr, pl.ds(0, L)])
                    nacc = jnp.abs(a_sl[0] - negrows_v[rr, pl.ds(0, L)])
                    for j in range(1, NJ):
                        pacc = pacc + jnp.abs(
                            a_sl[j] - posrows_v[rr, pl.ds(L * j, L)])
                        nacc = nacc + jnp.abs(
                            a_sl[j] - negrows_v[rr, pl.ds(L * j, L)])
                    sp = jnp.sum(pacc)
                    sn = jnp.sum(nacc)
                    tot = tot + jnp.maximum(
                        GAMMA + pcg[r] * sp - sn, jnp.float32(0.0))
                return tot

            total = plsc.parallel_loop(0, n_groups, carry=total)(group_body)

        outv[...] = jnp.where(lax.iota(jnp.int32, L) == 0, total,
                              jnp.float32(0.0))
        pltpu.sync_copy(outv, out_hbm.at[wid])

    return sc_kernel


def kernel(rel_emb, pconfi, rel_a, rel_pos, rel_neg):
    R = rel_pos.shape[0]
    sc = _build_sc(R)
    aidx = jnp.full((8,), rel_a, jnp.int32)
    partials = sc(rel_emb, pconfi, aidx,
                  rel_pos.astype(jnp.int32), rel_neg.astype(jnp.int32))
    return jnp.sum(partials)
